# baseline (speedup 1.0000x reference)
import jax
import jax.numpy as jnp
from jax import lax
from jax.experimental import pallas as pl
from jax.experimental.pallas import tpu as pltpu
from jax.experimental.pallas import tpu_sc as plsc

N_NODES = 10000
N_EDGES = 320000
D = 128
N_GRAPHS = 64

NC = 2
NS = 16
NW = NC * NS

CHUNK = 128
N_CHUNKS = N_EDGES // CHUNK
CPT = 80
WIN = 40
N_CHUNKS_PAD = 2520
ROWS_PAD = 10112
RPT = ROWS_PAD // NS

ROW_BLK = 2000
NBUF = 2


def _sc_segsum_body(x_hbm, src_hbm, dst_hbm, zeros_hbm, out_hbm,
                    src_v, dst_v, rows_v, accum, gsems):
    cid = lax.axis_index("c")
    sid = lax.axis_index("s")
    wid = sid * NC + cid
    cpt = jnp.minimum(CPT, N_CHUNKS - wid * CPT)

    pltpu.sync_copy(zeros_hbm, accum.at[pl.ds(sid * RPT, RPT)])
    plsc.subcore_barrier()

    for half in range(2):
        win1 = jnp.minimum(WIN, cpt)
        win = win1 if half == 0 else cpt - win1

        @pl.when(win > 0)
        def _process_window():
            start = wid * CPT + half * WIN
            pltpu.sync_copy(src_hbm.at[pl.ds(start, WIN)], src_v)
            pltpu.sync_copy(dst_hbm.at[pl.ds(start, WIN)], dst_v)

            for b in range(NBUF):
                pltpu.async_copy(x_hbm.at[src_v.at[b]], rows_v.at[b],
                                 gsems.at[b])

            def body(t, carry):
                base = t * NBUF
                for b in range(NBUF):
                    j = base + b

                    @pl.when(j < win)
                    def _():
                        pltpu.make_async_copy(
                            x_hbm.at[src_v.at[j]], rows_v.at[b],
                            gsems.at[b]).wait()
                        pltpu.sync_copy(rows_v.at[b],
                                        accum.at[dst_v.at[j]], add=True)

                        @pl.when(j + NBUF < win)
                        def _():
                            pltpu.async_copy(
                                x_hbm.at[src_v.at[j + NBUF]],
                                rows_v.at[b], gsems.at[b])
                return carry

            lax.fori_loop(0, (win + NBUF - 1) // NBUF, body, 0)

    plsc.subcore_barrier()
    pltpu.sync_copy(accum.at[pl.ds(sid * RPT, RPT)],
                    out_hbm.at[cid, pl.ds(sid * RPT, RPT)])


_sc_segsum = pl.kernel(
    _sc_segsum_body,
    out_type=jax.ShapeDtypeStruct((NC, ROWS_PAD, D), jnp.float32),
    mesh=plsc.VectorSubcoreMesh(core_axis_name="c", subcore_axis_name="s"),
    scratch_types=[
        pltpu.VMEM((WIN, CHUNK), jnp.int32),
        pltpu.VMEM((WIN, CHUNK), jnp.int32),
        pltpu.VMEM((NBUF, CHUNK, D), jnp.float32),
        pltpu.VMEM_SHARED((ROWS_PAD, D), jnp.float32),
        pltpu.SemaphoreType.DMA((NBUF,)),
    ],
)


def _tc_sage_body(a_ref, x_ref, wl_ref, wr_ref, bl_ref, o_ref):
    agg = a_ref[0] + a_ref[1]
    acc = jnp.dot(agg, wl_ref[...], preferred_element_type=jnp.float32)
    acc += jnp.dot(x_ref[...], wr_ref[...], preferred_element_type=jnp.float32)
    o_ref[...] = jnp.maximum(acc + bl_ref[...], 0.0)


def _tc_sage(a, x, wl, wr, bl):
    n = x.shape[0]
    grid = n // ROW_BLK
    return pl.pallas_call(
        _tc_sage_body,
        grid=(grid,),
        in_specs=[
            pl.BlockSpec((NC, ROW_BLK, D), lambda i: (0, i, 0)),
            pl.BlockSpec((ROW_BLK, D), lambda i: (i, 0)),
            pl.BlockSpec((D, D), lambda i: (0, 0)),
            pl.BlockSpec((D, D), lambda i: (0, 0)),
            pl.BlockSpec((1, D), lambda i: (0, 0)),
        ],
        out_specs=pl.BlockSpec((ROW_BLK, D), lambda i: (i, 0)),
        out_shape=jax.ShapeDtypeStruct((n, D), jnp.float32),
    )(a, x, wl, wr, bl.reshape(1, D))


def _tc_pool_body(h_ref, b_ref, w1_ref, b1_ref, w2_ref, b2_ref, o_ref):
    gids = lax.broadcasted_iota(jnp.int32, (N_GRAPHS, N_NODES), 0)
    onehot = (b_ref[...] == gids).astype(jnp.float32)
    sums = jnp.dot(onehot, h_ref[...], preferred_element_type=jnp.float32)
    counts = jnp.sum(onehot, axis=1, keepdims=True)
    pooled = sums / jnp.maximum(counts, 1.0)
    z = jnp.maximum(
        jnp.dot(pooled, w1_ref[...], preferred_element_type=jnp.float32)
        + b1_ref[...], 0.0)
    z = jnp.dot(z, w2_ref[...], preferred_element_type=jnp.float32) + b2_ref[...]
    o_ref[...] = 1.0 / (1.0 + jnp.exp(-z))


def _tc_pool(h, batch, w1, b1, w2, b2):
    return pl.pallas_call(
        _tc_pool_body,
        out_shape=jax.ShapeDtypeStruct((N_GRAPHS, 1), jnp.float32),
    )(h, batch.reshape(1, N_NODES), w1, b1.reshape(1, D),
      w2, b2.reshape(1, 1))


@jax.jit
def kernel(x, edge_index, batch, Wl1, bl1, Wr1, Wl2, bl2, Wr2, W1, b1, W2, b2):
    slack = jnp.zeros(((N_CHUNKS_PAD - N_CHUNKS) * CHUNK,), jnp.int32)
    srcp = jnp.concatenate([edge_index[0], slack]).reshape(N_CHUNKS_PAD, CHUNK)
    dstp = jnp.concatenate([edge_index[1], slack]).reshape(N_CHUNKS_PAD, CHUNK)
    zeros = jnp.zeros((RPT, D), jnp.float32)

    a1 = _sc_segsum(x, srcp, dstp, zeros)
    h1 = _tc_sage(a1, x, Wl1, Wr1, bl1)
    a2 = _sc_segsum(h1, srcp, dstp, zeros)
    h2 = _tc_sage(a2, h1, Wl2, Wr2, bl2)
    return _tc_pool(h2, batch, W1, b1, W2, b2)


# speedup vs baseline: 1.0287x; 1.0287x over previous
import jax
import jax.numpy as jnp
from jax import lax
from jax.experimental import pallas as pl
from jax.experimental.pallas import tpu as pltpu
from jax.experimental.pallas import tpu_sc as plsc

N_NODES = 10000
N_EDGES = 320000
D = 128
N_GRAPHS = 64

NC = 2
NS = 16
NW = NC * NS

CHUNK = 128
N_CHUNKS = N_EDGES // CHUNK
CPT = 80
WIN = 40
N_CHUNKS_PAD = 2520
ROWS_PAD = 10112
RPT = ROWS_PAD // NS

ROW_BLK = 2000
NBUF = 2


def _sc_segsum_body(x_hbm, e_hbm, zeros_hbm, out_hbm,
                    src_v, dst_v, rows_v, accum, gsems):
    cid = lax.axis_index("c")
    sid = lax.axis_index("s")
    wid = sid * NC + cid
    cpt = jnp.minimum(CPT, N_CHUNKS - wid * CPT)

    pltpu.sync_copy(zeros_hbm, accum.at[pl.ds(sid * RPT, RPT)])
    plsc.subcore_barrier()

    for half in range(2):
        win1 = jnp.minimum(WIN, cpt)
        win = win1 if half == 0 else cpt - win1

        @pl.when(win > 0)
        def _process_window():
            start = wid * CPT + half * WIN
            pltpu.sync_copy(e_hbm.at[0, pl.ds(start, WIN)], src_v)
            pltpu.sync_copy(e_hbm.at[1, pl.ds(start, WIN)], dst_v)

            for b in range(NBUF):
                pltpu.async_copy(x_hbm.at[src_v.at[b]], rows_v.at[b],
                                 gsems.at[b])

            def body(t, carry):
                base = t * NBUF
                for b in range(NBUF):
                    j = base + b

                    @pl.when(j < win)
                    def _():
                        pltpu.make_async_copy(
                            x_hbm.at[src_v.at[j]], rows_v.at[b],
                            gsems.at[b]).wait()
                        pltpu.sync_copy(rows_v.at[b],
                                        accum.at[dst_v.at[j]], add=True)

                        @pl.when(j + NBUF < win)
                        def _():
                            pltpu.async_copy(
                                x_hbm.at[src_v.at[j + NBUF]],
                                rows_v.at[b], gsems.at[b])
                return carry

            lax.fori_loop(0, (win + NBUF - 1) // NBUF, body, 0)

    plsc.subcore_barrier()
    pltpu.sync_copy(accum.at[pl.ds(sid * RPT, RPT)],
                    out_hbm.at[cid, pl.ds(sid * RPT, RPT)])


_sc_segsum = pl.kernel(
    _sc_segsum_body,
    out_type=jax.ShapeDtypeStruct((NC, ROWS_PAD, D), jnp.float32),
    mesh=plsc.VectorSubcoreMesh(core_axis_name="c", subcore_axis_name="s"),
    scratch_types=[
        pltpu.VMEM((WIN, CHUNK), jnp.int32),
        pltpu.VMEM((WIN, CHUNK), jnp.int32),
        pltpu.VMEM((NBUF, CHUNK, D), jnp.float32),
        pltpu.VMEM_SHARED((ROWS_PAD, D), jnp.float32),
        pltpu.SemaphoreType.DMA((NBUF,)),
    ],
)


def _tc_sage_body(a_ref, x_ref, wl_ref, wr_ref, bl_ref, o_ref):
    agg = a_ref[0] + a_ref[1]
    acc = jnp.dot(agg, wl_ref[...], preferred_element_type=jnp.float32)
    acc += jnp.dot(x_ref[...], wr_ref[...], preferred_element_type=jnp.float32)
    o_ref[...] = jnp.maximum(acc + bl_ref[...], 0.0)


def _tc_sage(a, x, wl, wr, bl):
    n = x.shape[0]
    grid = n // ROW_BLK
    return pl.pallas_call(
        _tc_sage_body,
        grid=(grid,),
        in_specs=[
            pl.BlockSpec((NC, ROW_BLK, D), lambda i: (0, i, 0)),
            pl.BlockSpec((ROW_BLK, D), lambda i: (i, 0)),
            pl.BlockSpec((D, D), lambda i: (0, 0)),
            pl.BlockSpec((D, D), lambda i: (0, 0)),
            pl.BlockSpec((1, D), lambda i: (0, 0)),
        ],
        out_specs=pl.BlockSpec((ROW_BLK, D), lambda i: (i, 0)),
        out_shape=jax.ShapeDtypeStruct((n, D), jnp.float32),
    )(a, x, wl, wr, bl.reshape(1, D))


def _tc_pool_body(h_ref, b_ref, w1_ref, b1_ref, w2_ref, b2_ref, o_ref):
    gids = lax.broadcasted_iota(jnp.int32, (N_GRAPHS, N_NODES), 0)
    onehot = (b_ref[...] == gids).astype(jnp.float32)
    sums = jnp.dot(onehot, h_ref[...], preferred_element_type=jnp.float32)
    counts = jnp.sum(onehot, axis=1, keepdims=True)
    pooled = sums / jnp.maximum(counts, 1.0)
    z = jnp.maximum(
        jnp.dot(pooled, w1_ref[...], preferred_element_type=jnp.float32)
        + b1_ref[...], 0.0)
    z = jnp.dot(z, w2_ref[...], preferred_element_type=jnp.float32) + b2_ref[...]
    o_ref[...] = 1.0 / (1.0 + jnp.exp(-z))


def _tc_pool(h, batch, w1, b1, w2, b2):
    return pl.pallas_call(
        _tc_pool_body,
        out_shape=jax.ShapeDtypeStruct((N_GRAPHS, 1), jnp.float32),
    )(h, batch.reshape(1, N_NODES), w1, b1.reshape(1, D),
      w2, b2.reshape(1, 1))


@jax.jit
def kernel(x, edge_index, batch, Wl1, bl1, Wr1, Wl2, bl2, Wr2, W1, b1, W2, b2):
    epad = jnp.pad(
        edge_index, ((0, 0), (0, (N_CHUNKS_PAD - N_CHUNKS) * CHUNK))
    ).reshape(2, N_CHUNKS_PAD, CHUNK)
    zeros = jnp.zeros((RPT, D), jnp.float32)

    a1 = _sc_segsum(x, epad, zeros)
    h1 = _tc_sage(a1, x, Wl1, Wr1, bl1)
    a2 = _sc_segsum(h1, epad, zeros)
    h2 = _tc_sage(a2, h1, Wl2, Wr2, bl2)
    return _tc_pool(h2, batch, W1, b1, W2, b2)
